# baseline (speedup 1.0000x reference)
import functools

import jax
import jax.numpy as jnp
from jax import lax
from jax.experimental import pallas as pl
from jax.experimental.pallas import tpu as pltpu
from jax.experimental.pallas import tpu_sc as plsc


@functools.cache
def _make_sc_passthrough(b: int):
    info = plsc.get_sparse_core_info()
    nc, ns = info.num_cores, info.num_subcores
    nw = nc * ns
    b_per_w = b // nw
    mesh = plsc.VectorSubcoreMesh(core_axis_name="c", subcore_axis_name="s")

    @functools.partial(
        pl.kernel,
        mesh=mesh,
        out_type=jax.ShapeDtypeStruct((b,), jnp.int32),
        scratch_types=[pltpu.VMEM((b_per_w,), jnp.int32)],
    )
    def sc_kernel(idx_hbm, out_hbm, idx_v):
        wid = lax.axis_index("s") * nc + lax.axis_index("c")
        base = wid * b_per_w
        pltpu.sync_copy(idx_hbm.at[pl.ds(base, b_per_w)], idx_v)
        pltpu.sync_copy(idx_v, out_hbm.at[pl.ds(base, b_per_w)])

    return sc_kernel


def _tc_fused(xr, idx, alphas_pad, n_neurons):
    bsz, n_sf, cs = xr.shape
    npad = alphas_pad.shape[0]
    bb = 8
    nbuf = 8
    nchunks = bsz // bb
    idx2 = idx.reshape(bsz, 1)

    def body(x_hbm, idx_ref, a_ref, o_ref, att_ref, buf, sems):
        ids_flat = idx_ref[...]
        iota_v = lax.broadcasted_iota(jnp.int32, (bsz, npad), 1)
        onehot = jnp.where(iota_v == ids_flat, 1.0, 0.0)
        gathered = jnp.dot(onehot, a_ref[...],
                           preferred_element_type=jnp.float32)
        row_max = jnp.max(gathered, axis=1, keepdims=True)
        e = jnp.exp(gathered - row_max)
        att = e / jnp.sum(e, axis=1, keepdims=True)
        att_ref[...] = att

        def start(c, slot):
            pltpu.make_async_copy(
                x_hbm.at[pl.ds(c * bb, bb)], buf.at[slot], sems.at[slot]
            ).start()

        def wait(slot):
            pltpu.make_async_copy(
                x_hbm.at[pl.ds(0, bb)], buf.at[slot], sems.at[slot]
            ).wait()

        for s in range(nbuf):
            start(s, s)

        def outer(g, carry):
            base = g * nbuf
            for s in range(nbuf):
                c = base + s
                wait(s)
                w_blk = att_ref[pl.ds(c * bb, bb)]
                o_ref[pl.ds(c * bb, bb)] = jnp.sum(
                    buf[s] * w_blk[:, :, None], axis=1)
                nxt = c + nbuf

                @pl.when(nxt < nchunks)
                def _():
                    start(nxt, s)
            return carry

        lax.fori_loop(0, nchunks // nbuf, outer, 0)

    return pl.pallas_call(
        body,
        in_specs=[
            pl.BlockSpec(memory_space=pl.ANY),
            pl.BlockSpec(memory_space=pltpu.VMEM),
            pl.BlockSpec(memory_space=pltpu.VMEM),
        ],
        out_specs=[
            pl.BlockSpec(memory_space=pltpu.VMEM),
            pl.BlockSpec(memory_space=pltpu.VMEM),
        ],
        out_shape=[
            jax.ShapeDtypeStruct((bsz, cs), jnp.float32),
            jax.ShapeDtypeStruct((bsz, n_sf), jnp.float32),
        ],
        scratch_shapes=[
            pltpu.VMEM((nbuf, bb, n_sf, cs), jnp.float32),
            pltpu.SemaphoreType.DMA((nbuf,)),
        ],
    )(xr, idx2, alphas_pad)


def kernel(x, neuron_list, alphas):
    b, n, c, s = x.shape
    xr = x.reshape(b, n, c * s)
    n_neurons, n_sf = alphas.shape
    npad = 64
    alphas_pad = jnp.zeros((npad, n_sf), jnp.float32).at[:n_neurons].set(alphas)
    idx_sc = _make_sc_passthrough(b)(neuron_list)
    out, alphas_att = _tc_fused(xr, idx_sc, alphas_pad, n_neurons)
    return out, alphas_att


# speedup vs baseline: 1.0493x; 1.0493x over previous
import functools

import jax
import jax.numpy as jnp
from jax import lax
from jax.experimental import pallas as pl
from jax.experimental.pallas import tpu as pltpu
from jax.experimental.pallas import tpu_sc as plsc


@functools.cache
def _make_sc_passthrough(b: int):
    info = plsc.get_sparse_core_info()
    nc, ns = info.num_cores, info.num_subcores
    nw = nc * ns
    b_per_w = b // nw
    mesh = plsc.VectorSubcoreMesh(core_axis_name="c", subcore_axis_name="s")

    @functools.partial(
        pl.kernel,
        mesh=mesh,
        out_type=jax.ShapeDtypeStruct((b,), jnp.int32),
        scratch_types=[pltpu.VMEM((b_per_w,), jnp.int32)],
    )
    def sc_kernel(idx_hbm, out_hbm, idx_v):
        wid = lax.axis_index("s") * nc + lax.axis_index("c")
        base = wid * b_per_w
        pltpu.sync_copy(idx_hbm.at[pl.ds(base, b_per_w)], idx_v)
        pltpu.sync_copy(idx_v, out_hbm.at[pl.ds(base, b_per_w)])

    return sc_kernel


def _tc_fused(xr, idx, alphas_pad, n_neurons):
    bsz, n_sf, cs = xr.shape
    npad = alphas_pad.shape[0]
    bb = 8
    nbuf = 8
    nchunks = bsz // bb
    idx2 = idx.reshape(bsz, 1)

    def body(x_hbm, idx_ref, a_ref, o_ref, att_ref, buf, sems):
        def start(c, slot):
            pltpu.make_async_copy(
                x_hbm.at[pl.ds(c * bb, bb)], buf.at[slot], sems.at[slot]
            ).start()

        def wait(slot):
            pltpu.make_async_copy(
                x_hbm.at[pl.ds(0, bb)], buf.at[slot], sems.at[slot]
            ).wait()

        for s in range(nbuf):
            start(s, s)

        ids_flat = idx_ref[...]
        iota_v = lax.broadcasted_iota(jnp.int32, (bsz, npad), 1)
        onehot = jnp.where(iota_v == ids_flat, 1.0, 0.0)
        gathered = jnp.dot(onehot, a_ref[...],
                           preferred_element_type=jnp.float32)
        row_max = jnp.max(gathered, axis=1, keepdims=True)
        e = jnp.exp(gathered - row_max)
        att = e / jnp.sum(e, axis=1, keepdims=True)
        att_ref[...] = att

        def outer(g, carry):
            base = g * nbuf
            for s in range(nbuf):
                c = base + s
                wait(s)
                w_blk = att_ref[pl.ds(c * bb, bb)]
                o_ref[pl.ds(c * bb, bb)] = jnp.sum(
                    buf[s] * w_blk[:, :, None], axis=1)
                nxt = c + nbuf

                @pl.when(nxt < nchunks)
                def _():
                    start(nxt, s)
            return carry

        lax.fori_loop(0, nchunks // nbuf, outer, 0)

    return pl.pallas_call(
        body,
        in_specs=[
            pl.BlockSpec(memory_space=pl.ANY),
            pl.BlockSpec(memory_space=pltpu.VMEM),
            pl.BlockSpec(memory_space=pltpu.VMEM),
        ],
        out_specs=[
            pl.BlockSpec(memory_space=pltpu.VMEM),
            pl.BlockSpec(memory_space=pltpu.VMEM),
        ],
        out_shape=[
            jax.ShapeDtypeStruct((bsz, cs), jnp.float32),
            jax.ShapeDtypeStruct((bsz, n_sf), jnp.float32),
        ],
        scratch_shapes=[
            pltpu.VMEM((nbuf, bb, n_sf, cs), jnp.float32),
            pltpu.SemaphoreType.DMA((nbuf,)),
        ],
    )(xr, idx2, alphas_pad)


def kernel(x, neuron_list, alphas):
    b, n, c, s = x.shape
    xr = x.reshape(b, n, c * s)
    n_neurons, n_sf = alphas.shape
    npad = 64
    alphas_pad = jnp.zeros((npad, n_sf), jnp.float32).at[:n_neurons].set(alphas)
    out, alphas_att = _tc_fused(xr, neuron_list, alphas_pad, n_neurons)
    return out, alphas_att


# speedup vs baseline: 3.8512x; 3.6704x over previous
import functools

import jax
import jax.numpy as jnp
from jax import lax
from jax.experimental import pallas as pl
from jax.experimental.pallas import tpu as pltpu


def _tc_fused(xt, idx, alphas_pad):
    bsz, c_dim, s_dim, n_sf = xt.shape
    cs = c_dim * s_dim
    npad = alphas_pad.shape[0]
    bb = 8
    nbuf = 8
    nchunks = bsz // bb

    def body(x_hbm, idx_ref, a_ref, o_ref, att_ref, buf, sems):
        def start(c, slot):
            pltpu.make_async_copy(
                x_hbm.at[pl.ds(c * bb, bb)], buf.at[slot], sems.at[slot]
            ).start()

        def wait(slot):
            pltpu.make_async_copy(
                x_hbm.at[pl.ds(0, bb)], buf.at[slot], sems.at[slot]
            ).wait()

        for s in range(nbuf):
            start(s, s)

        ids_flat = idx_ref[...]
        iota_v = lax.broadcasted_iota(jnp.int32, (bsz, npad), 1)
        onehot = jnp.where(iota_v == ids_flat, 1.0, 0.0)
        gathered = jnp.dot(onehot, a_ref[...],
                           preferred_element_type=jnp.float32)
        row_max = jnp.max(gathered, axis=1, keepdims=True)
        e = jnp.exp(gathered - row_max)
        att = e / jnp.sum(e, axis=1, keepdims=True)
        att_ref[...] = att

        def outer(g, carry):
            base = g * nbuf
            for s in range(nbuf):
                c = base + s
                wait(s)
                for b in range(bb):
                    row = c * bb + b
                    xb = buf[s, b].reshape(cs, n_sf)
                    w_row = att_ref[pl.ds(row, 1), :]
                    o_ref[pl.ds(row, 1), :] = lax.dot_general(
                        w_row, xb, (((1,), (1,)), ((), ())),
                        preferred_element_type=jnp.float32)
                nxt = c + nbuf

                @pl.when(nxt < nchunks)
                def _():
                    start(nxt, s)
            return carry

        lax.fori_loop(0, nchunks // nbuf, outer, 0)

    return pl.pallas_call(
        body,
        in_specs=[
            pl.BlockSpec(memory_space=pl.ANY),
            pl.BlockSpec(memory_space=pltpu.VMEM),
            pl.BlockSpec(memory_space=pltpu.VMEM),
        ],
        out_specs=[
            pl.BlockSpec(memory_space=pltpu.VMEM),
            pl.BlockSpec(memory_space=pltpu.VMEM),
        ],
        out_shape=[
            jax.ShapeDtypeStruct((bsz, cs), jnp.float32),
            jax.ShapeDtypeStruct((bsz, n_sf), jnp.float32),
        ],
        scratch_shapes=[
            pltpu.VMEM((nbuf, bb, c_dim, s_dim, n_sf), jnp.float32),
            pltpu.SemaphoreType.DMA((nbuf,)),
        ],
    )(xt, idx, alphas_pad)


def kernel(x, neuron_list, alphas):
    n_neurons, n_sf = alphas.shape
    npad = 64
    alphas_pad = jnp.zeros((npad, n_sf), jnp.float32).at[:n_neurons].set(alphas)
    xt = x.transpose(0, 2, 3, 1)
    idx2 = neuron_list.reshape(x.shape[0], 1)
    out, alphas_att = _tc_fused(xt, idx2, alphas_pad)
    return out, alphas_att
